# baseline (device time: 121811 ns/iter reference)
import jax
import jax.numpy as jnp
from jax import lax
from jax.experimental import pallas as pl
from jax.experimental.pallas import tpu as pltpu

N_DEV = 4
SEQ = 1024
SEQ_PER = 256
D_MODEL = 1024
H_PER = 8
DH = 128
SCALE = 0.08838834764831843
F32 = jnp.float32


def kernel(x, Wq, Wo, Wk, Wv):
    def body(x_ref, wq_ref, wo_ref, wk_ref, wv_ref, out_ref,
             ag_buf, p_ref, rs_send, rs_recv,
             ag_send_sems, ag_recv_sems, rs_send_sems, rs_recv_sems):
        my = lax.axis_index("i")
        right = lax.rem(my + 1, N_DEV)
        left = lax.rem(my + N_DEV - 1, N_DEV)

        barrier = pltpu.get_barrier_semaphore()
        for nbr in (left, right):
            pl.semaphore_signal(
                barrier, inc=1,
                device_id=(nbr,), device_id_type=pl.DeviceIdType.MESH,
            )
        pl.semaphore_wait(barrier, 2)

        ag_buf[pl.ds(my * SEQ_PER, SEQ_PER), :] = x_ref[0, :, :]
        for h in range(N_DEV - 1):
            origin = lax.rem(my - h + N_DEV, N_DEV)
            sl = pl.ds(origin * SEQ_PER, SEQ_PER)
            rdma = pltpu.make_async_remote_copy(
                src_ref=ag_buf.at[sl, :],
                dst_ref=ag_buf.at[sl, :],
                send_sem=ag_send_sems.at[h],
                recv_sem=ag_recv_sems.at[h],
                device_id=(right,),
                device_id_type=pl.DeviceIdType.MESH,
            )
            rdma.start()
            rdma.wait()

        x_full = ag_buf[:, :]
        q = jnp.dot(x_full, wq_ref[:, :], preferred_element_type=F32)
        k = jnp.dot(x_full, wk_ref[:, :], preferred_element_type=F32)
        v = jnp.dot(x_full, wv_ref[:, :], preferred_element_type=F32)

        partial = jnp.zeros((SEQ, D_MODEL), F32)
        for h in range(H_PER):
            qh = q[:, h * DH:(h + 1) * DH]
            kh = k[:, h * DH:(h + 1) * DH]
            vh = v[:, h * DH:(h + 1) * DH]
            s = lax.dot_general(
                qh, kh, (((1,), (1,)), ((), ())),
                preferred_element_type=F32,
            ) * SCALE
            m = jnp.max(s, axis=1, keepdims=True)
            p = jnp.exp(s - m)
            l = jnp.sum(p, axis=1, keepdims=True)
            oh = jnp.dot(p, vh, preferred_element_type=F32) / l
            partial = partial + jnp.dot(
                oh, wo_ref[h * DH:(h + 1) * DH, :], preferred_element_type=F32
            )
        p_ref[:, :] = partial

        for h in range(N_DEV - 1):
            c = lax.rem(my - h - 1 + 2 * N_DEV, N_DEV)
            chunk = p_ref[pl.ds(c * SEQ_PER, SEQ_PER), :]
            if h == 0:
                rs_send[h, :, :] = chunk
            else:
                rs_send[h, :, :] = chunk + rs_recv[h - 1, :, :]
            rdma = pltpu.make_async_remote_copy(
                src_ref=rs_send.at[h],
                dst_ref=rs_recv.at[h],
                send_sem=rs_send_sems.at[h],
                recv_sem=rs_recv_sems.at[h],
                device_id=(right,),
                device_id_type=pl.DeviceIdType.MESH,
            )
            rdma.start()
            rdma.wait()

        out_ref[0, :, :] = (
            rs_recv[N_DEV - 2, :, :] + p_ref[pl.ds(my * SEQ_PER, SEQ_PER), :]
        )

    return pl.pallas_call(
        body,
        out_shape=jax.ShapeDtypeStruct((1, SEQ_PER, D_MODEL), F32),
        in_specs=[pl.BlockSpec(memory_space=pltpu.VMEM)] * 5,
        out_specs=pl.BlockSpec(memory_space=pltpu.VMEM),
        scratch_shapes=[
            pltpu.VMEM((SEQ, D_MODEL), F32),
            pltpu.VMEM((SEQ, D_MODEL), F32),
            pltpu.VMEM((N_DEV - 1, SEQ_PER, D_MODEL), F32),
            pltpu.VMEM((N_DEV - 1, SEQ_PER, D_MODEL), F32),
            pltpu.SemaphoreType.DMA((N_DEV - 1,)),
            pltpu.SemaphoreType.DMA((N_DEV - 1,)),
            pltpu.SemaphoreType.DMA((N_DEV - 1,)),
            pltpu.SemaphoreType.DMA((N_DEV - 1,)),
        ],
        compiler_params=pltpu.CompilerParams(collective_id=0),
    )(x, Wq, Wo, Wk, Wv)


# device time: 101077 ns/iter; 1.2051x vs baseline; 1.2051x over previous
import jax
import jax.numpy as jnp
from jax import lax
from jax.experimental import pallas as pl
from jax.experimental.pallas import tpu as pltpu

N_DEV = 4
SEQ = 1024
SEQ_PER = 256
D_MODEL = 1024
H_PER = 8
DH = 128
SCALE = 0.08838834764831843
F32 = jnp.float32


def kernel(x, Wq, Wo, Wk, Wv):
    def body(x_ref, wq_ref, wo_ref, wk_ref, wv_ref, out_ref,
             ag_buf, q_ref, k_ref, v_ref, rs_send, rs_recv,
             ag_send_sems, ag_recv_sems, rs_send_sems, rs_recv_sems):
        my = lax.axis_index("i")
        right = lax.rem(my + 1, N_DEV)
        left = lax.rem(my + N_DEV - 1, N_DEV)

        barrier = pltpu.get_barrier_semaphore()
        for nbr in (left, right):
            pl.semaphore_signal(
                barrier, inc=1,
                device_id=(nbr,), device_id_type=pl.DeviceIdType.MESH,
            )
        pl.semaphore_wait(barrier, 2)

        def qkv_chunk(c):
            sl = pl.ds(c * SEQ_PER, SEQ_PER)
            xc = ag_buf[sl, :]
            q_ref[sl, :] = jnp.dot(xc, wq_ref[:, :], preferred_element_type=F32)
            k_ref[sl, :] = jnp.dot(xc, wk_ref[:, :], preferred_element_type=F32)
            v_ref[sl, :] = jnp.dot(xc, wv_ref[:, :], preferred_element_type=F32)

        ag_buf[pl.ds(my * SEQ_PER, SEQ_PER), :] = x_ref[0, :, :]
        for h in range(N_DEV - 1):
            origin = lax.rem(my - h + N_DEV, N_DEV)
            sl = pl.ds(origin * SEQ_PER, SEQ_PER)
            rdma = pltpu.make_async_remote_copy(
                src_ref=ag_buf.at[sl, :],
                dst_ref=ag_buf.at[sl, :],
                send_sem=ag_send_sems.at[h],
                recv_sem=ag_recv_sems.at[h],
                device_id=(right,),
                device_id_type=pl.DeviceIdType.MESH,
            )
            rdma.start()
            qkv_chunk(origin)
            rdma.wait_recv()
            rdma.wait_send()
        qkv_chunk(lax.rem(my + 1, N_DEV))

        def attn_chunk(c):
            qsl = pl.ds(c * SEQ_PER, SEQ_PER)
            acc = jnp.zeros((SEQ_PER, D_MODEL), F32)
            for h in range(H_PER):
                hs = pl.ds(h * DH, DH)
                qh = q_ref[qsl, hs]
                kh = k_ref[:, hs]
                vh = v_ref[:, hs]
                s = lax.dot_general(
                    qh, kh, (((1,), (1,)), ((), ())),
                    preferred_element_type=F32,
                ) * SCALE
                m = jnp.max(s, axis=1, keepdims=True)
                p = jnp.exp(s - m)
                l = jnp.sum(p, axis=1, keepdims=True)
                oh = jnp.dot(p, vh, preferred_element_type=F32) / l
                acc = acc + jnp.dot(
                    oh, wo_ref[hs, :], preferred_element_type=F32
                )
            return acc

        rdmas = []
        for h in range(N_DEV - 1):
            c = lax.rem(my - h - 1 + 2 * N_DEV, N_DEV)
            part = attn_chunk(c)
            if h > 0:
                rdmas[h - 1].wait_recv()
                part = part + rs_recv[h - 1, :, :]
            rs_send[h, :, :] = part
            rdma = pltpu.make_async_remote_copy(
                src_ref=rs_send.at[h],
                dst_ref=rs_recv.at[h],
                send_sem=rs_send_sems.at[h],
                recv_sem=rs_recv_sems.at[h],
                device_id=(right,),
                device_id_type=pl.DeviceIdType.MESH,
            )
            rdma.start()
            rdmas.append(rdma)

        final_part = attn_chunk(my)
        rdmas[N_DEV - 2].wait_recv()
        out_ref[0, :, :] = rs_recv[N_DEV - 2, :, :] + final_part
        for r in rdmas:
            r.wait_send()

    return pl.pallas_call(
        body,
        out_shape=jax.ShapeDtypeStruct((1, SEQ_PER, D_MODEL), F32),
        in_specs=[pl.BlockSpec(memory_space=pltpu.VMEM)] * 5,
        out_specs=pl.BlockSpec(memory_space=pltpu.VMEM),
        scratch_shapes=[
            pltpu.VMEM((SEQ, D_MODEL), F32),
            pltpu.VMEM((SEQ, D_MODEL), F32),
            pltpu.VMEM((SEQ, D_MODEL), F32),
            pltpu.VMEM((SEQ, D_MODEL), F32),
            pltpu.VMEM((N_DEV - 1, SEQ_PER, D_MODEL), F32),
            pltpu.VMEM((N_DEV - 1, SEQ_PER, D_MODEL), F32),
            pltpu.SemaphoreType.DMA((N_DEV - 1,)),
            pltpu.SemaphoreType.DMA((N_DEV - 1,)),
            pltpu.SemaphoreType.DMA((N_DEV - 1,)),
            pltpu.SemaphoreType.DMA((N_DEV - 1,)),
        ],
        compiler_params=pltpu.CompilerParams(collective_id=0),
    )(x, Wq, Wo, Wk, Wv)


# device time: 82293 ns/iter; 1.4802x vs baseline; 1.2283x over previous
import jax
import jax.numpy as jnp
from jax import lax
from jax.experimental import pallas as pl
from jax.experimental.pallas import tpu as pltpu

N_DEV = 4
SEQ = 1024
SEQ_PER = 256
D_MODEL = 1024
H_PER = 8
DH = 128
SCALE = 0.08838834764831843
F32 = jnp.float32


def kernel(x, Wq, Wo, Wk, Wv):
    def body(x_ref, wq_ref, wo_ref, wk_ref, wv_ref, out_ref,
             ag_buf, q_ref, k_ref, v_ref, rs_send, rs_recv,
             ag_send_sems, ag_recv_sems, rs_send_sems, rs_recv_sems):
        my = lax.axis_index("i")
        right = lax.rem(my + 1, N_DEV)
        left = lax.rem(my + N_DEV - 1, N_DEV)

        def m4(c):
            return lax.rem(c + 2 * N_DEV, N_DEV)

        barrier = pltpu.get_barrier_semaphore()
        for nbr in (left, right):
            pl.semaphore_signal(
                barrier, inc=1,
                device_id=(nbr,), device_id_type=pl.DeviceIdType.MESH,
            )
        pl.semaphore_wait(barrier, 2)

        def ag_copy(origin, slot, dst):
            sl = pl.ds(origin * SEQ_PER, SEQ_PER)
            return pltpu.make_async_remote_copy(
                src_ref=ag_buf.at[sl, :],
                dst_ref=ag_buf.at[sl, :],
                send_sem=ag_send_sems.at[slot],
                recv_sem=ag_recv_sems.at[slot],
                device_id=(dst,),
                device_id_type=pl.DeviceIdType.MESH,
            )

        def qkv_chunk(c):
            sl = pl.ds(c * SEQ_PER, SEQ_PER)
            xc = ag_buf[sl, :]
            q_ref[sl, :] = jnp.dot(xc, wq_ref[:, :], preferred_element_type=F32)
            k_ref[sl, :] = jnp.dot(xc, wk_ref[:, :], preferred_element_type=F32)
            v_ref[sl, :] = jnp.dot(xc, wv_ref[:, :], preferred_element_type=F32)

        ag_buf[pl.ds(my * SEQ_PER, SEQ_PER), :] = x_ref[0, :, :]
        ag_r = ag_copy(my, 0, right)
        ag_l = ag_copy(my, 1, left)
        ag_r.start()
        ag_l.start()
        qkv_chunk(my)
        ag_r.wait_recv()
        ag_f = ag_copy(m4(my - 1), 2, right)
        ag_f.start()
        qkv_chunk(m4(my - 1))
        ag_l.wait_recv()
        qkv_chunk(m4(my + 1))
        ag_f.wait_recv()
        qkv_chunk(m4(my + 2))
        ag_r.wait_send()
        ag_l.wait_send()
        ag_f.wait_send()

        def attn_chunk(c):
            qsl = pl.ds(c * SEQ_PER, SEQ_PER)
            acc = jnp.zeros((SEQ_PER, D_MODEL), F32)
            for h in range(H_PER):
                hs = pl.ds(h * DH, DH)
                qh = q_ref[qsl, hs]
                kh = k_ref[:, hs]
                vh = v_ref[:, hs]
                s = lax.dot_general(
                    qh, kh, (((1,), (1,)), ((), ())),
                    preferred_element_type=F32,
                ) * SCALE
                m = jnp.max(s, axis=1, keepdims=True)
                p = jnp.exp(s - m)
                l = jnp.sum(p, axis=1, keepdims=True)
                oh = jnp.dot(p, vh, preferred_element_type=F32) / l
                acc = acc + jnp.dot(
                    oh, wo_ref[hs, :], preferred_element_type=F32
                )
            return acc

        def rs_copy(slot, dst):
            return pltpu.make_async_remote_copy(
                src_ref=rs_send.at[slot],
                dst_ref=rs_recv.at[slot],
                send_sem=rs_send_sems.at[slot],
                recv_sem=rs_recv_sems.at[slot],
                device_id=(dst,),
                device_id_type=pl.DeviceIdType.MESH,
            )

        rs_send[0, :, :] = attn_chunk(m4(my + 1))
        rs0 = rs_copy(0, right)
        rs0.start()
        rs_send[1, :, :] = attn_chunk(m4(my - 2))
        rs1 = rs_copy(1, left)
        rs1.start()
        pc = attn_chunk(m4(my - 1))
        rs1.wait_recv()
        rs_send[2, :, :] = pc + rs_recv[1, :, :]
        rs2 = rs_copy(2, left)
        rs2.start()
        own = attn_chunk(my)
        rs0.wait_recv()
        rs2.wait_recv()
        out_ref[0, :, :] = own + rs_recv[0, :, :] + rs_recv[2, :, :]
        rs0.wait_send()
        rs1.wait_send()
        rs2.wait_send()

    return pl.pallas_call(
        body,
        out_shape=jax.ShapeDtypeStruct((1, SEQ_PER, D_MODEL), F32),
        in_specs=[pl.BlockSpec(memory_space=pltpu.VMEM)] * 5,
        out_specs=pl.BlockSpec(memory_space=pltpu.VMEM),
        scratch_shapes=[
            pltpu.VMEM((SEQ, D_MODEL), F32),
            pltpu.VMEM((SEQ, D_MODEL), F32),
            pltpu.VMEM((SEQ, D_MODEL), F32),
            pltpu.VMEM((SEQ, D_MODEL), F32),
            pltpu.VMEM((3, SEQ_PER, D_MODEL), F32),
            pltpu.VMEM((3, SEQ_PER, D_MODEL), F32),
            pltpu.SemaphoreType.DMA((3,)),
            pltpu.SemaphoreType.DMA((3,)),
            pltpu.SemaphoreType.DMA((3,)),
            pltpu.SemaphoreType.DMA((3,)),
        ],
        compiler_params=pltpu.CompilerParams(collective_id=0),
    )(x, Wq, Wo, Wk, Wv)


# device time: 61489 ns/iter; 1.9810x vs baseline; 1.3383x over previous
import jax
import jax.numpy as jnp
from jax import lax
from jax.experimental import pallas as pl
from jax.experimental.pallas import tpu as pltpu

N_DEV = 4
SEQ = 1024
SEQ_PER = 256
D_MODEL = 1024
H_PER = 8
DH = 128
SCALE = 0.08838834764831843
F32 = jnp.float32
BF16 = jnp.bfloat16


def kernel(x, Wq, Wo, Wk, Wv):
    def body(x_ref, wq_ref, wo_ref, wk_ref, wv_ref, out_ref,
             ag_buf, q_ref, k_ref, v_ref, rs_send, rs_recv,
             ag_send_sems, ag_recv_sems, rs_send_sems, rs_recv_sems):
        my = lax.axis_index("i")
        right = lax.rem(my + 1, N_DEV)
        left = lax.rem(my + N_DEV - 1, N_DEV)

        def m4(c):
            return lax.rem(c + 2 * N_DEV, N_DEV)

        barrier = pltpu.get_barrier_semaphore()
        for nbr in (left, right):
            pl.semaphore_signal(
                barrier, inc=1,
                device_id=(nbr,), device_id_type=pl.DeviceIdType.MESH,
            )
        pl.semaphore_wait(barrier, 2)

        wq_bf = wq_ref[:, :].astype(BF16)
        wk_bf = wk_ref[:, :].astype(BF16)
        wv_bf = wv_ref[:, :].astype(BF16)
        wo_bf = wo_ref[:, :].astype(BF16)

        def ag_copy(origin, slot, dst):
            sl = pl.ds(origin * SEQ_PER, SEQ_PER)
            return pltpu.make_async_remote_copy(
                src_ref=ag_buf.at[sl, :],
                dst_ref=ag_buf.at[sl, :],
                send_sem=ag_send_sems.at[slot],
                recv_sem=ag_recv_sems.at[slot],
                device_id=(dst,),
                device_id_type=pl.DeviceIdType.MESH,
            )

        def qkv_chunk(c):
            sl = pl.ds(c * SEQ_PER, SEQ_PER)
            xc = ag_buf[sl, :]
            q_ref[sl, :] = jnp.dot(
                xc, wq_bf, preferred_element_type=F32).astype(BF16)
            k_ref[sl, :] = jnp.dot(
                xc, wk_bf, preferred_element_type=F32).astype(BF16)
            v_ref[sl, :] = jnp.dot(
                xc, wv_bf, preferred_element_type=F32).astype(BF16)

        ag_buf[pl.ds(my * SEQ_PER, SEQ_PER), :] = x_ref[0, :, :].astype(BF16)
        ag_r = ag_copy(my, 0, right)
        ag_l = ag_copy(my, 1, left)
        ag_r.start()
        ag_l.start()
        qkv_chunk(my)
        ag_r.wait_recv()
        ag_f = ag_copy(m4(my - 1), 2, right)
        ag_f.start()
        qkv_chunk(m4(my - 1))
        ag_l.wait_recv()
        qkv_chunk(m4(my + 1))
        ag_f.wait_recv()
        qkv_chunk(m4(my + 2))
        ag_r.wait_send()
        ag_l.wait_send()
        ag_f.wait_send()

        def attn_chunk(c):
            qsl = pl.ds(c * SEQ_PER, SEQ_PER)
            acc = jnp.zeros((SEQ_PER, D_MODEL), F32)
            for h in range(H_PER):
                hs = pl.ds(h * DH, DH)
                qh = q_ref[qsl, hs]
                kh = k_ref[:, hs]
                vh = v_ref[:, hs]
                s = lax.dot_general(
                    qh, kh, (((1,), (1,)), ((), ())),
                    preferred_element_type=F32,
                ) * SCALE
                m = jnp.max(s, axis=1, keepdims=True)
                p = jnp.exp(s - m)
                l = jnp.sum(p, axis=1, keepdims=True)
                oh = jnp.dot(
                    (p / l).astype(BF16), vh, preferred_element_type=F32)
                acc = acc + jnp.dot(
                    oh.astype(BF16), wo_bf[h * DH:(h + 1) * DH, :],
                    preferred_element_type=F32,
                )
            return acc

        def rs_copy(slot, dst):
            return pltpu.make_async_remote_copy(
                src_ref=rs_send.at[slot],
                dst_ref=rs_recv.at[slot],
                send_sem=rs_send_sems.at[slot],
                recv_sem=rs_recv_sems.at[slot],
                device_id=(dst,),
                device_id_type=pl.DeviceIdType.MESH,
            )

        rs_send[0, :, :] = attn_chunk(m4(my + 1)).astype(BF16)
        rs0 = rs_copy(0, right)
        rs0.start()
        rs_send[1, :, :] = attn_chunk(m4(my - 2)).astype(BF16)
        rs1 = rs_copy(1, left)
        rs1.start()
        pc = attn_chunk(m4(my - 1))
        rs1.wait_recv()
        rs_send[2, :, :] = (
            pc + rs_recv[1, :, :].astype(F32)).astype(BF16)
        rs2 = rs_copy(2, left)
        rs2.start()
        own = attn_chunk(my)
        rs0.wait_recv()
        rs2.wait_recv()
        out_ref[0, :, :] = (
            own
            + rs_recv[0, :, :].astype(F32)
            + rs_recv[2, :, :].astype(F32)
        )
        rs0.wait_send()
        rs1.wait_send()
        rs2.wait_send()

    return pl.pallas_call(
        body,
        out_shape=jax.ShapeDtypeStruct((1, SEQ_PER, D_MODEL), F32),
        in_specs=[pl.BlockSpec(memory_space=pltpu.VMEM)] * 5,
        out_specs=pl.BlockSpec(memory_space=pltpu.VMEM),
        scratch_shapes=[
            pltpu.VMEM((SEQ, D_MODEL), BF16),
            pltpu.VMEM((SEQ, D_MODEL), BF16),
            pltpu.VMEM((SEQ, D_MODEL), BF16),
            pltpu.VMEM((SEQ, D_MODEL), BF16),
            pltpu.VMEM((3, SEQ_PER, D_MODEL), BF16),
            pltpu.VMEM((3, SEQ_PER, D_MODEL), BF16),
            pltpu.SemaphoreType.DMA((3,)),
            pltpu.SemaphoreType.DMA((3,)),
            pltpu.SemaphoreType.DMA((3,)),
            pltpu.SemaphoreType.DMA((3,)),
        ],
        compiler_params=pltpu.CompilerParams(collective_id=0),
    )(x, Wq, Wo, Wk, Wv)


# device time: 53343 ns/iter; 2.2835x vs baseline; 1.1527x over previous
import functools

import jax
import jax.numpy as jnp
from jax import lax
from jax.experimental import pallas as pl
from jax.experimental.pallas import tpu as pltpu

N_DEV = 4
SEQ = 1024
SEQ_PER = 256
D_MODEL = 1024
H_PER = 8
DH = 128
SCALE = 0.08838834764831843
F32 = jnp.float32
BF16 = jnp.bfloat16


def kernel(x, Wq, Wo, Wk, Wv):
    def body(x_ref, wq_ref, wo_ref, wk_ref, wv_ref, out_ref,
             ag_buf, q_ref, k_ref, v_ref, rs_send, rs_recv,
             ag_send_sems, ag_recv_sems, rs_send_sems, rs_recv_sems):
        my = lax.axis_index("i")
        right = lax.rem(my + 1, N_DEV)
        left = lax.rem(my + N_DEV - 1, N_DEV)
        opp = lax.rem(my + 2, N_DEV)

        def m4(c):
            return lax.rem(c + 2 * N_DEV, N_DEV)

        barrier = pltpu.get_barrier_semaphore()
        for nbr in (left, right):
            pl.semaphore_signal(
                barrier, inc=1,
                device_id=(nbr,), device_id_type=pl.DeviceIdType.MESH,
            )
        pl.semaphore_wait(barrier, 2)

        wq_bf = wq_ref[:, :].astype(BF16)
        wk_bf = wk_ref[:, :].astype(BF16)
        wv_bf = wv_ref[:, :].astype(BF16)
        wo_bf = wo_ref[:, :].astype(BF16)

        def ag_copy(origin, slot, dst):
            sl = pl.ds(origin * SEQ_PER, SEQ_PER)
            return pltpu.make_async_remote_copy(
                src_ref=ag_buf.at[sl, :],
                dst_ref=ag_buf.at[sl, :],
                send_sem=ag_send_sems.at[slot],
                recv_sem=ag_recv_sems.at[slot],
                device_id=(dst,),
                device_id_type=pl.DeviceIdType.MESH,
            )

        def qkv_chunk(c):
            sl = pl.ds(c * SEQ_PER, SEQ_PER)
            xc = ag_buf[sl, :]
            q_ref[sl, :] = jnp.dot(
                xc, wq_bf, preferred_element_type=F32).astype(BF16)
            k_ref[sl, :] = jnp.dot(
                xc, wk_bf, preferred_element_type=F32).astype(BF16)
            v_ref[sl, :] = jnp.dot(
                xc, wv_bf, preferred_element_type=F32).astype(BF16)

        ag_buf[pl.ds(my * SEQ_PER, SEQ_PER), :] = x_ref[0, :, :].astype(BF16)
        ag_r = ag_copy(my, 0, right)
        ag_l = ag_copy(my, 1, left)
        ag_r.start()
        ag_l.start()
        qkv_chunk(my)
        ag_r.wait_recv()
        ag_f = ag_copy(m4(my - 1), 2, right)
        ag_f.start()
        qkv_chunk(m4(my - 1))
        ag_l.wait_recv()
        qkv_chunk(m4(my + 1))
        ag_f.wait_recv()
        qkv_chunk(m4(my + 2))
        ag_r.wait_send()
        ag_l.wait_send()
        ag_f.wait_send()

        def attn_chunk(c):
            qsl = pl.ds(c * SEQ_PER, SEQ_PER)
            acc = jnp.zeros((SEQ_PER, D_MODEL), F32)
            for h in range(H_PER):
                hs = pl.ds(h * DH, DH)
                qh = q_ref[qsl, hs]
                kh = k_ref[:, hs]
                vh = v_ref[:, hs]
                s = lax.dot_general(
                    qh, kh, (((1,), (1,)), ((), ())),
                    preferred_element_type=F32,
                ) * SCALE
                p = jnp.exp(s)
                l = jnp.sum(p, axis=1, keepdims=True)
                oh = jnp.dot(
                    p.astype(BF16), vh, preferred_element_type=F32) / l
                acc = acc + jnp.dot(
                    oh.astype(BF16), wo_bf[h * DH:(h + 1) * DH, :],
                    preferred_element_type=F32,
                )
            return acc

        def rs_copy(slot, dst):
            return pltpu.make_async_remote_copy(
                src_ref=rs_send.at[slot],
                dst_ref=rs_recv.at[slot],
                send_sem=rs_send_sems.at[slot],
                recv_sem=rs_recv_sems.at[slot],
                device_id=(dst,),
                device_id_type=pl.DeviceIdType.MESH,
            )

        rs_send[2, :, :] = attn_chunk(opp).astype(BF16)
        rs2 = rs_copy(2, opp)
        rs2.start()
        rs_send[0, :, :] = attn_chunk(m4(my + 1)).astype(BF16)
        rs0 = rs_copy(0, right)
        rs0.start()
        rs_send[1, :, :] = attn_chunk(m4(my - 1)).astype(BF16)
        rs1 = rs_copy(1, left)
        rs1.start()
        own = attn_chunk(my)
        rs0.wait_recv()
        rs1.wait_recv()
        rs2.wait_recv()
        out_ref[0, :, :] = (
            own
            + rs_recv[0, :, :].astype(F32)
            + rs_recv[1, :, :].astype(F32)
            + rs_recv[2, :, :].astype(F32)
        )
        rs0.wait_send()
        rs1.wait_send()
        rs2.wait_send()

        @functools.partial(
            pl.run_scoped, second_barrier=pltpu.SemaphoreType.REGULAR)
        def _(second_barrier):
            for nbr in (left, right):
                pl.semaphore_signal(
                    second_barrier, inc=1,
                    device_id=(nbr,), device_id_type=pl.DeviceIdType.MESH,
                )
            pl.semaphore_wait(second_barrier, 2)

    return pl.pallas_call(
        body,
        out_shape=jax.ShapeDtypeStruct((1, SEQ_PER, D_MODEL), F32),
        in_specs=[pl.BlockSpec(memory_space=pltpu.VMEM)] * 5,
        out_specs=pl.BlockSpec(memory_space=pltpu.VMEM),
        scratch_shapes=[
            pltpu.VMEM((SEQ, D_MODEL), BF16),
            pltpu.VMEM((SEQ, D_MODEL), BF16),
            pltpu.VMEM((SEQ, D_MODEL), BF16),
            pltpu.VMEM((SEQ, D_MODEL), BF16),
            pltpu.VMEM((3, SEQ_PER, D_MODEL), BF16),
            pltpu.VMEM((3, SEQ_PER, D_MODEL), BF16),
            pltpu.SemaphoreType.DMA((3,)),
            pltpu.SemaphoreType.DMA((3,)),
            pltpu.SemaphoreType.DMA((3,)),
            pltpu.SemaphoreType.DMA((3,)),
        ],
        compiler_params=pltpu.CompilerParams(collective_id=0),
    )(x, Wq, Wo, Wk, Wv)
